# baseline (speedup 1.0000x reference)
import jax
import jax.numpy as jnp
from jax.experimental import pallas as pl
from jax.experimental.pallas import tpu as pltpu

_T = 1024
_OUT = 1000
_N = 192


def _dott(w_ref, B):
    return jax.lax.dot_general(w_ref[...], B, (((0,), (0,)), ((), ())),
                               preferred_element_type=jnp.float32)


def _pair_weight(m, wo, c, keep, s):
    k, wc, n = m.shape
    core = m.reshape(k * wc, n)
    m3 = core.reshape(k * wc, wo, c)
    wp = keep // (2 * c)
    core = jnp.concatenate([m3[:, 0:2 * wp:2], m3[:, 1:2 * wp:2]],
                           axis=1).reshape(k * wc, keep)
    even = jnp.pad(core, ((0, s), (0, 0)))
    odd = jnp.pad(core, ((s, 0), (0, 0)))
    return jnp.concatenate([even, odd], axis=1)


def _conv_pool(B, m_ref, b_ref):
    acc = _dott(m_ref, B)
    s = acc.shape[0] // 2
    m0 = jnp.maximum(acc[:s], acc[s:])
    m1 = jnp.maximum(m0[:s // 2], m0[s // 2:])
    return (m1 + b_ref[...]).astype(jnp.bfloat16)


def _pair_windows(P, hin, k, pad, hp, t, wc):
    zero = jnp.zeros((wc, t), jnp.bfloat16)

    def row(j):
        return P[:, j * t:(j + 1) * t] if 0 <= j < hin else zero

    wins = [jnp.concatenate([row(2 * i - pad + d) for d in range(k + 1)],
                            axis=0) for i in range(hp)]
    return jnp.concatenate(wins, axis=1)


def _fwd_kernel(x_ref,
                m1, b1, m2, b2, m3, b3, m4, b4,
                fw1, fb1, fw2, fb2,
                out_ref):
    t = x_ref.shape[1]
    xb = x_ref[...].astype(jnp.bfloat16)
    z64 = jnp.zeros((64, t), jnp.bfloat16)

    wins = [jnp.concatenate([z64, xb[0:128]], axis=0)]
    wins += [xb[64 * i - 64: 64 * i + 128] for i in range(1, 15)]
    wins += [jnp.concatenate([xb[896:1024], z64], axis=0)]
    B1 = jnp.concatenate(wins, axis=1)
    P1 = _conv_pool(B1, m1, b1)

    B2 = _pair_windows(P1, 16, 5, 0, 6, t, 96)
    P2 = _conv_pool(B2, m2, b2)

    B3 = _pair_windows(P2, 6, 3, 1, 3, t, 96)
    P3 = _conv_pool(B3, m3, b3)

    B4 = _pair_windows(P3, 3, 3, 1, 1, t, 96)
    f = _conv_pool(B4, m4, b4)

    h = _dott(fw1, f) + fb1[...]
    y = _dott(fw2, h.astype(jnp.bfloat16)) + fb2[...]
    out_ref[...] = y[:_OUT]


def _const_specs(arrays):
    return [pl.BlockSpec(a.shape, lambda i, _nd=a.ndim: (0,) * _nd)
            for a in arrays]


def kernel(c1_m, c1_b, c1_rsel, c1_csel,
           c2_m, c2_b, c2_rsel, c2_csel,
           c3_m, c3_b, c3_rsel, c3_csel,
           c4_m, c4_b, c4_rsel, c4_csel,
           fc1_w, fc1_b, fc2_w, fc2_b,
           x):
    n = x.shape[0]
    xt = x.reshape(n, 32 * 32).T
    n_pad = ((n + _T - 1) // _T) * _T
    if n_pad != n:
        xt = jnp.concatenate(
            [xt, jnp.zeros((32 * 32, n_pad - n), xt.dtype)], axis=1)

    consts = [
        _pair_weight(c1_m, 32, 6, _N, 32), c1_b[:, :96].T,
        _pair_weight(c2_m, 12, 16, _N, 96), c2_b[:, :96].T,
        _pair_weight(c3_m, 6, 32, _N, 96), c3_b[:, :96].T,
        _pair_weight(c4_m, 3, 64, 128, 96), c4_b[:, :64].T,
        fc1_w, fc1_b.T, fc2_w, fc2_b.T,
    ]
    weight_bytes = sum(int(a.size) * a.dtype.itemsize for a in consts)

    out = pl.pallas_call(
        _fwd_kernel,
        out_shape=jax.ShapeDtypeStruct((_OUT, n_pad), jnp.float32),
        grid=(n_pad // _T,),
        in_specs=[pl.BlockSpec((32 * 32, _T), lambda i: (0, i))]
                 + _const_specs(consts),
        out_specs=pl.BlockSpec((_OUT, _T), lambda i: (0, i)),
        compiler_params=pltpu.CompilerParams(
            dimension_semantics=("parallel",),
            vmem_limit_bytes=64 * 1024 * 1024),
        cost_estimate=pl.CostEstimate(
            flops=7_500_000 * n_pad,
            transcendentals=0,
            bytes_accessed=weight_bytes + n_pad * (32 * 32 * 4 + _OUT * 4)),
    )(xt, *consts)
    return out.T[:n]


# speedup vs baseline: 1.1168x; 1.1168x over previous
import jax
import jax.numpy as jnp
from jax.experimental import pallas as pl
from jax.experimental.pallas import tpu as pltpu

_T = 1024
_OUT = 1000
_N = 192


def _dott(w_ref, B):
    return jax.lax.dot_general(w_ref[...], B, (((0,), (0,)), ((), ())),
                               preferred_element_type=jnp.float32)


def _pool_perm(m2d, wo, c, keep):
    m3 = m2d.reshape(m2d.shape[0], wo, c)
    wp = keep // (2 * c)
    ev = m3[:, 0:2 * wp:2]
    od = m3[:, 1:2 * wp:2]
    return jnp.concatenate([ev, od], axis=1).reshape(m2d.shape[0], keep)


def _conv_pool(B, m_ref, b_ref, hp, t):
    acc = _dott(m_ref, B)
    m0 = jnp.maximum(acc[:, :hp * t], acc[:, hp * t:])
    s = m0.shape[0] // 2
    m1 = jnp.maximum(m0[:s], m0[s:])
    return (m1 + b_ref[...]).astype(jnp.bfloat16)


def _windows(P, hin, k, pad, hp, t, wc):
    zero = jnp.zeros((wc, t), jnp.bfloat16)

    def row(j):
        return P[:, j * t:(j + 1) * t] if 0 <= j < hin else zero

    def win(i):
        return jnp.concatenate([row(i - pad + d) for d in range(k)], axis=0)

    order = [2 * i for i in range(hp)] + [2 * i + 1 for i in range(hp)]
    return jnp.concatenate([win(i) for i in order], axis=1)


def _fwd_kernel(x_ref,
                m1, b1, m2, b2, m3, b3, m4, b4,
                fw1, fb1, fw2, fb2,
                out_ref):
    t = x_ref.shape[1]
    xb = x_ref[...].astype(jnp.bfloat16)
    z64 = jnp.zeros((64, t), jnp.bfloat16)

    wins = [jnp.concatenate([z64, xb[0:128]], axis=0)]
    wins += [xb[64 * i - 64: 64 * i + 128] for i in range(1, 15)]
    wins += [jnp.concatenate([xb[896:1024], z64], axis=0)]
    B1 = jnp.concatenate(wins, axis=1)
    acc = _dott(m1, B1)
    m0 = jnp.maximum(acc[:_N], acc[_N:])
    mm = jnp.maximum(m0[:96], m0[96:])
    P1 = (mm + b1[...]).astype(jnp.bfloat16)

    B2 = _windows(P1, 16, 5, 0, 6, t, 96)
    P2 = _conv_pool(B2, m2, b2, 6, t)

    B3 = _windows(P2, 6, 3, 1, 3, t, 96)
    P3 = _conv_pool(B3, m3, b3, 3, t)

    B4 = _windows(P3, 3, 3, 1, 1, t, 96)
    f = _conv_pool(B4, m4, b4, 1, t)

    h = _dott(fw1, f) + fb1[...]
    y = _dott(fw2, h.astype(jnp.bfloat16)) + fb2[...]
    out_ref[...] = y[:_OUT]


def _const_specs(arrays):
    return [pl.BlockSpec(a.shape, lambda i, _nd=a.ndim: (0,) * _nd)
            for a in arrays]


def kernel(c1_m, c1_b, c1_rsel, c1_csel,
           c2_m, c2_b, c2_rsel, c2_csel,
           c3_m, c3_b, c3_rsel, c3_csel,
           c4_m, c4_b, c4_rsel, c4_csel,
           fc1_w, fc1_b, fc2_w, fc2_b,
           x):
    n = x.shape[0]
    xt = x.reshape(n, 32 * 32).T
    n_pad = ((n + _T - 1) // _T) * _T
    if n_pad != n:
        xt = jnp.concatenate(
            [xt, jnp.zeros((32 * 32, n_pad - n), xt.dtype)], axis=1)

    c1_core = _pool_perm(c1_m.reshape(160, _N), 32, 6, _N)
    c1_pair = jnp.concatenate(
        [jnp.pad(c1_core, ((0, 32), (0, 0))),
         jnp.pad(c1_core, ((32, 0), (0, 0)))], axis=1)

    consts = [
        c1_pair, c1_b[:, :96].T,
        _pool_perm(c2_m.reshape(480, _N), 12, 16, _N), c2_b[:, :96].T,
        _pool_perm(c3_m.reshape(288, _N), 6, 32, _N), c3_b[:, :96].T,
        _pool_perm(c4_m.reshape(288, _N), 3, 64, 128), c4_b[:, :64].T,
        fc1_w, fc1_b.T, fc2_w, fc2_b.T,
    ]
    weight_bytes = sum(int(a.size) * a.dtype.itemsize for a in consts)

    out = pl.pallas_call(
        _fwd_kernel,
        out_shape=jax.ShapeDtypeStruct((_OUT, n_pad), jnp.float32),
        grid=(n_pad // _T,),
        in_specs=[pl.BlockSpec((32 * 32, _T), lambda i: (0, i))]
                 + _const_specs(consts),
        out_specs=pl.BlockSpec((_OUT, _T), lambda i: (0, i)),
        compiler_params=pltpu.CompilerParams(
            dimension_semantics=("parallel",),
            vmem_limit_bytes=64 * 1024 * 1024),
        cost_estimate=pl.CostEstimate(
            flops=7_500_000 * n_pad,
            transcendentals=0,
            bytes_accessed=weight_bytes + n_pad * (32 * 32 * 4 + _OUT * 4)),
    )(xt, *consts)
    return out.T[:n]
